# baseline (speedup 1.0000x reference)
import functools

import jax
import jax.numpy as jnp
from jax import lax
from jax.experimental import pallas as pl
from jax.experimental.pallas import tpu as pltpu
from jax.experimental.pallas import tpu_sc as plsc

_GROUPS = 8
_NUM_VARS = 64
_VAR_DIM = 64
_PROJ_DIM = 32

_BLK = 512

_info = plsc.get_sparse_core_info()
_NC = _info.num_cores
_NS = _info.num_subcores
_NW = _NC * _NS
_CHUNK = 128


def _logits_argmax_body(x_ref, wpre_ref, bpre_ref, wwp_ref, bwp_ref, idx_ref):
    x = x_ref[...]
    h = jnp.dot(x, wpre_ref[...], preferred_element_type=jnp.float32)
    h = h + bpre_ref[...]
    logits = jnp.dot(h, wwp_ref[...], preferred_element_type=jnp.float32)
    logits = logits + bwp_ref[...]
    cols = []
    for g in range(_GROUPS):
        sub = logits[:, g * _NUM_VARS:(g + 1) * _NUM_VARS]
        k = jnp.argmax(sub, axis=-1).astype(jnp.int32) + g * _NUM_VARS
        cols.append(k[:, None])
    idx_ref[...] = jnp.concatenate(cols, axis=-1)


def _postproject_body(q_ref, wpost_ref, bpost_ref, out_ref):
    out = jnp.dot(q_ref[...], wpost_ref[...], preferred_element_type=jnp.float32)
    out_ref[...] = out + bpost_ref[...]


def _make_sc_gather(rows, nch):
    mesh = plsc.VectorSubcoreMesh(core_axis_name="c", subcore_axis_name="s")

    @functools.partial(
        pl.kernel,
        mesh=mesh,
        compiler_params=pltpu.CompilerParams(use_tc_tiling_on_sc=False),
        out_type=jax.ShapeDtypeStruct((rows, _VAR_DIM), jnp.float32),
        scratch_types=[
            pltpu.VMEM((nch, _CHUNK), jnp.int32),
            pltpu.VMEM((_CHUNK, _VAR_DIM), jnp.float32),
            pltpu.VMEM((_CHUNK, _VAR_DIM), jnp.float32),
            pltpu.SemaphoreType.DMA,
            pltpu.SemaphoreType.DMA,
        ],
    )
    def gather(idx_hbm, cb_hbm, out_hbm, idx_v, buf0, buf1, sem0, sem1):
        wid = lax.axis_index("s") * _NC + lax.axis_index("c")
        pltpu.sync_copy(idx_hbm.at[wid], idx_v)
        bufs = (buf0, buf1)
        sems = (sem0, sem1)
        base = wid * nch * _CHUNK
        cps = [None] * nch
        cps[0] = pltpu.async_copy(cb_hbm.at[idx_v.at[0]], bufs[0], sems[0])
        for j in range(nch):
            if j + 1 < nch:
                cps[j + 1] = pltpu.async_copy(
                    cb_hbm.at[idx_v.at[j + 1]], bufs[(j + 1) % 2], sems[(j + 1) % 2])
            cps[j].wait()
            pltpu.sync_copy(bufs[j % 2],
                            out_hbm.at[pl.ds(base + j * _CHUNK, _CHUNK)])

    return gather


def kernel(x, W_pre, b_pre, W_wp, b_wp, codebook, W_post, b_post):
    B, T, IN_DIM = x.shape
    OUT_DIM = W_post.shape[1]
    BT = B * T
    xf = x.reshape(BT, IN_DIM)

    idx = pl.pallas_call(
        _logits_argmax_body,
        grid=(BT // _BLK,),
        in_specs=[
            pl.BlockSpec((_BLK, IN_DIM), lambda i: (i, 0)),
            pl.BlockSpec((IN_DIM, _PROJ_DIM), lambda i: (0, 0)),
            pl.BlockSpec((1, _PROJ_DIM), lambda i: (0, 0)),
            pl.BlockSpec((_PROJ_DIM, _GROUPS * _NUM_VARS), lambda i: (0, 0)),
            pl.BlockSpec((1, _GROUPS * _NUM_VARS), lambda i: (0, 0)),
        ],
        out_specs=pl.BlockSpec((_BLK, _GROUPS), lambda i: (i, 0)),
        out_shape=jax.ShapeDtypeStruct((BT, _GROUPS), jnp.int32),
    )(xf, W_pre, b_pre.reshape(1, -1), W_wp, b_wp.reshape(1, -1))

    rows = BT * _GROUPS
    nch = rows // (_NW * _CHUNK)
    idx3 = idx.reshape(_NW, nch, _CHUNK)
    q = _make_sc_gather(rows, nch)(idx3, codebook)

    out = pl.pallas_call(
        _postproject_body,
        grid=(BT // _BLK,),
        in_specs=[
            pl.BlockSpec((_BLK, _GROUPS * _VAR_DIM), lambda i: (i, 0)),
            pl.BlockSpec((_GROUPS * _VAR_DIM, OUT_DIM), lambda i: (0, 0)),
            pl.BlockSpec((1, OUT_DIM), lambda i: (0, 0)),
        ],
        out_specs=pl.BlockSpec((_BLK, OUT_DIM), lambda i: (i, 0)),
        out_shape=jax.ShapeDtypeStruct((BT, OUT_DIM), jnp.float32),
    )(q.reshape(BT, _GROUPS * _VAR_DIM), W_post, b_post.reshape(1, -1))

    return out.reshape(B, T, OUT_DIM)
